# baseline (speedup 1.0000x reference)
import functools

import jax
import jax.numpy as jnp
from jax import lax
from jax.experimental import pallas as pl
from jax.experimental.pallas import tpu as pltpu
from jax.experimental.pallas import tpu_sc as plsc

_VOCAB = 1000000
_MAXLEN = 200
_EMBED = 64
_BATCH = 4096

_NC = 2
_NS = 16
_NW = _NC * _NS
_SEQ_PER_W = _BATCH // _NW
_K = 4
_CHUNK_ROWS = _K * _MAXLEN
_N_CHUNKS = _SEQ_PER_W // _K
_IDX_COLS = 100
_IDX_ROWS = _CHUNK_ROWS // _IDX_COLS


def _emb_body(x_hbm, tok_hbm, pos_hbm, out_hbm, idx_v, rows_v, pos_v, gsem):
    wid = lax.axis_index("s") * _NC + lax.axis_index("c")

    pltpu.sync_copy(pos_hbm, pos_v)

    def chunk_body(g, carry):
        row0 = pl.multiple_of((wid * _SEQ_PER_W + g * _K) * _MAXLEN, 8)
        ir0 = pl.multiple_of(row0 // _IDX_COLS, 8)
        pltpu.sync_copy(x_hbm.at[pl.ds(ir0, _IDX_ROWS), :], idx_v)
        copies = []
        for j in range(_IDX_ROWS):
            copies.append(pltpu.async_copy(
                tok_hbm.at[idx_v.at[j]],
                rows_v.at[pl.ds(j * _IDX_COLS, _IDX_COLS), :],
                gsem))
        for c in copies:
            c.wait()

        def pos_body(p, _):
            pv = [pos_v[p, pl.ds(q * 16, 16)] for q in range(4)]
            for s in range(_K):
                r = s * _MAXLEN + p
                for q in range(4):
                    sl = pl.ds(q * 16, 16)
                    rows_v[r, sl] = rows_v[r, sl] + pv[q]
            return 0

        lax.fori_loop(0, _MAXLEN, pos_body, 0)

        pltpu.sync_copy(rows_v, out_hbm.at[pl.ds(row0, _CHUNK_ROWS), :])
        return carry

    lax.fori_loop(0, _N_CHUNKS, chunk_body, 0)


@jax.jit
def _emb(x2, token_table, pos_table):
    mesh = plsc.VectorSubcoreMesh(core_axis_name="c", subcore_axis_name="s")
    f = functools.partial(
        pl.kernel,
        out_type=jax.ShapeDtypeStruct((_BATCH * _MAXLEN, _EMBED), jnp.float32),
        mesh=mesh,
        compiler_params=pltpu.CompilerParams(use_tc_tiling_on_sc=False),
        scratch_types=[
            pltpu.VMEM((_IDX_ROWS, _IDX_COLS), jnp.int32),
            pltpu.VMEM((_CHUNK_ROWS, _EMBED), jnp.float32),
            pltpu.VMEM((_MAXLEN, _EMBED), jnp.float32),
            pltpu.SemaphoreType.DMA,
        ],
    )(_emb_body)
    return f(x2, token_table, pos_table)


def kernel(x, token_table, pos_table):
    x2 = x.astype(jnp.int32).reshape(_BATCH * _MAXLEN // _IDX_COLS, _IDX_COLS)
    out = _emb(x2, token_table, pos_table)
    return out.reshape(_BATCH, _MAXLEN, _EMBED)


# speedup vs baseline: 1.0818x; 1.0818x over previous
import functools

import jax
import jax.numpy as jnp
from jax import lax
from jax.experimental import pallas as pl
from jax.experimental.pallas import tpu as pltpu
from jax.experimental.pallas import tpu_sc as plsc

_VOCAB = 1000000
_MAXLEN = 200
_EMBED = 64
_BATCH = 4096

_NC = 2
_NS = 16
_NW = _NC * _NS
_SEQ_PER_W = _BATCH // _NW
_K = 2
_CHUNK_ROWS = _K * _MAXLEN
_N_CHUNKS = _SEQ_PER_W // _K
_IDX_COLS = 100
_IDX_ROWS = _CHUNK_ROWS // _IDX_COLS
_NSLOT = 4


def _emb_body(x_hbm, tok_hbm, pos_hbm, out_hbm, idx_v, rows_v, pos_v,
              gsem, isem, osem):
    wid = lax.axis_index("s") * _NC + lax.axis_index("c")
    row_base = wid * _SEQ_PER_W * _MAXLEN

    def chunk_row0(g):
        return pl.multiple_of(row_base + g * _CHUNK_ROWS, 8)

    def idx_fire(g, u):
        ir0 = pl.multiple_of((row_base + g * _CHUNK_ROWS) // _IDX_COLS, 4)
        return pltpu.async_copy(
            x_hbm.at[pl.ds(ir0, _IDX_ROWS), :], idx_v[u], isem[u])

    def gather_fire(g, t):
        for j in range(_IDX_ROWS):
            pltpu.async_copy(
                tok_hbm.at[idx_v[t].at[j]],
                rows_v[t].at[pl.ds(j * _IDX_COLS, _IDX_COLS), :],
                gsem[t])

    def gather_wait(t):
        for j in range(_IDX_ROWS):
            pltpu.make_async_copy(
                tok_hbm.at[idx_v[t].at[j]],
                rows_v[t].at[pl.ds(j * _IDX_COLS, _IDX_COLS), :],
                gsem[t]).wait()

    def wb_fire(g, t):
        pltpu.async_copy(
            rows_v[t], out_hbm.at[pl.ds(chunk_row0(g), _CHUNK_ROWS), :],
            osem[t])

    def wb_wait(g, t):
        pltpu.make_async_copy(
            rows_v[t], out_hbm.at[pl.ds(chunk_row0(g), _CHUNK_ROWS), :],
            osem[t]).wait()

    def add_pos(t):
        rows = rows_v[t]

        def pbody(p, carry):
            for q in range(4):
                sl = pl.ds(q * 16, 16)
                pv = pos_v[p, sl]
                for s in range(_K):
                    plsc.addupdate(rows.at[s * _MAXLEN + p, sl], pv)
            return carry

        lax.fori_loop(0, _MAXLEN, pbody, 0)

    pltpu.sync_copy(pos_hbm, pos_v)

    idx_fire(0, 0).wait()
    idx_fire(1, 1).wait()
    gather_fire(0, 0)
    gather_fire(1, 1)
    idx_fire(2, 2)

    def body(i, carry):
        for s in range(_NSLOT):
            g = i * _NSLOT + s
            t = (s + 2) % _NSLOT
            u = (s + 3) % _NSLOT
            if s == 0:
                idx_fire(g + 3, u)
            else:
                @pl.when(i < (_N_CHUNKS // _NSLOT) - 1)
                def _():
                    idx_fire(g + 3, u)
            if s < 2:
                @pl.when(i > 0)
                def _():
                    wb_wait(g - 2, t)
                pltpu.make_async_copy(
                    x_hbm.at[pl.ds(0, _IDX_ROWS), :], idx_v[t],
                    isem[t]).wait()
                gather_fire(g + 2, t)
            else:
                @pl.when(i < (_N_CHUNKS // _NSLOT) - 1)
                def _():
                    wb_wait(g - 2, t)
                    pltpu.make_async_copy(
                        x_hbm.at[pl.ds(0, _IDX_ROWS), :], idx_v[t],
                        isem[t]).wait()
                    gather_fire(g + 2, t)
            gather_wait(s)
            add_pos(s)
            wb_fire(g, s)
        return carry

    lax.fori_loop(0, _N_CHUNKS // _NSLOT, body, 0)

    for s in range(_NSLOT):
        wb_wait(_N_CHUNKS - _NSLOT + s, s)


@jax.jit
def _emb(x2, token_table, pos_table):
    mesh = plsc.VectorSubcoreMesh(core_axis_name="c", subcore_axis_name="s")
    f = functools.partial(
        pl.kernel,
        out_type=jax.ShapeDtypeStruct((_BATCH * _MAXLEN, _EMBED), jnp.float32),
        mesh=mesh,
        compiler_params=pltpu.CompilerParams(use_tc_tiling_on_sc=False),
        scratch_types=[
            [pltpu.VMEM((_IDX_ROWS, _IDX_COLS), jnp.int32)] * _NSLOT,
            [pltpu.VMEM((_CHUNK_ROWS, _EMBED), jnp.float32)] * _NSLOT,
            pltpu.VMEM((_MAXLEN, _EMBED), jnp.float32),
            [pltpu.SemaphoreType.DMA] * _NSLOT,
            [pltpu.SemaphoreType.DMA] * _NSLOT,
            [pltpu.SemaphoreType.DMA] * _NSLOT,
        ],
    )(_emb_body)
    return f(x2, token_table, pos_table)


def kernel(x, token_table, pos_table):
    x2 = x.astype(jnp.int32).reshape(_BATCH * _MAXLEN // _IDX_COLS, _IDX_COLS)
    out = _emb(x2, token_table, pos_table)
    return out.reshape(_BATCH, _MAXLEN, _EMBED)


# speedup vs baseline: 1.3227x; 1.2227x over previous
import functools

import jax
import jax.numpy as jnp
from jax import lax
from jax.experimental import pallas as pl
from jax.experimental.pallas import tpu as pltpu
from jax.experimental.pallas import tpu_sc as plsc

_VOCAB = 1000000
_MAXLEN = 200
_EMBED = 64
_BATCH = 4096

_NC = 2
_NS = 16
_NW = _NC * _NS
_SEQ_PER_W = _BATCH // _NW
_CHUNK_ROWS = _MAXLEN
_N_CHUNKS = _SEQ_PER_W
_SPLITS = ((0, 128), (128, 72))
_NSLOT = 2
_NIDX = 4


def _emb_body(x_hbm, tok_hbm, pos_hbm, out_hbm, idx_v, rows_v, obuf_v, pos_v,
              gsem, isem, osem):
    wid = lax.axis_index("s") * _NC + lax.axis_index("c")
    row_base = wid * _SEQ_PER_W * _MAXLEN

    def chunk_row0(g):
        return pl.multiple_of(row_base + g * _CHUNK_ROWS, 8)

    def idx_fire(g, u):
        return pltpu.async_copy(
            x_hbm.at[pl.ds(chunk_row0(g), _CHUNK_ROWS)], idx_v[u], isem[u])

    def idx_wait(u):
        pltpu.make_async_copy(
            x_hbm.at[pl.ds(0, _CHUNK_ROWS)], idx_v[u], isem[u]).wait()

    def gather_fire(u, t):
        for (o, n) in _SPLITS:
            pltpu.async_copy(
                tok_hbm.at[idx_v[u].at[pl.ds(o, n)]],
                rows_v[t].at[pl.ds(o, n), :],
                gsem[t])

    def gather_wait(u, t):
        for (o, n) in _SPLITS:
            pltpu.make_async_copy(
                tok_hbm.at[idx_v[u].at[pl.ds(o, n)]],
                rows_v[t].at[pl.ds(o, n), :],
                gsem[t]).wait()

    def wb_fire(g, t):
        pltpu.async_copy(
            obuf_v[t], out_hbm.at[pl.ds(chunk_row0(g), _CHUNK_ROWS), :],
            osem[t])

    def wb_wait(g, t):
        pltpu.make_async_copy(
            obuf_v[t], out_hbm.at[pl.ds(chunk_row0(g), _CHUNK_ROWS), :],
            osem[t]).wait()

    def add_pos(t):
        rows = rows_v[t]
        ob = obuf_v[t]

        def pbody(p, carry):
            for q in range(4):
                sl = pl.ds(q * 16, 16)
                ob[p, sl] = rows[p, sl] + pos_v[p, sl]
            return carry

        lax.fori_loop(0, _MAXLEN, pbody, 0)

    pltpu.sync_copy(pos_hbm, pos_v)

    idx_fire(0, 0).wait()
    idx_fire(1, 1).wait()
    gather_fire(0, 0)
    gather_fire(1, 1)
    idx_fire(2, 2)
    idx_fire(3, 3)

    n_outer = _N_CHUNKS // _NIDX

    def body(i, carry):
        for s in range(_NIDX):
            g = i * _NIDX + s
            rs = s % _NSLOT
            gather_wait(s, rs)
            if s < _NSLOT:
                @pl.when(i > 0)
                def _():
                    wb_wait(g - _NSLOT, rs)
            else:
                wb_wait(g - _NSLOT, rs)
            add_pos(rs)
            if s < _NSLOT:
                idx_wait((s + _NSLOT) % _NIDX)
                gather_fire((s + _NSLOT) % _NIDX, rs)
            else:
                @pl.when(i < n_outer - 1)
                def _():
                    idx_wait((s + _NSLOT) % _NIDX)
                    gather_fire((s + _NSLOT) % _NIDX, rs)
            @pl.when(i < n_outer - 1)
            def _():
                idx_fire(g + _NIDX, s)
            wb_fire(g, rs)
        return carry

    lax.fori_loop(0, n_outer, body, 0)

    for s in range(_NSLOT):
        wb_wait(_N_CHUNKS - _NSLOT + s, s)


@jax.jit
def _emb(xf, tok128, pos_table):
    mesh = plsc.VectorSubcoreMesh(core_axis_name="c", subcore_axis_name="s")
    f = functools.partial(
        pl.kernel,
        out_type=jax.ShapeDtypeStruct((_BATCH * _MAXLEN, _EMBED), jnp.float32),
        mesh=mesh,
        compiler_params=pltpu.CompilerParams(use_tc_tiling_on_sc=True),
        scratch_types=[
            [pltpu.VMEM((_CHUNK_ROWS,), jnp.int32)] * _NIDX,
            [pltpu.VMEM((_CHUNK_ROWS, 2 * _EMBED), jnp.float32)] * _NSLOT,
            [pltpu.VMEM((_CHUNK_ROWS, _EMBED), jnp.float32)] * _NSLOT,
            pltpu.VMEM((_MAXLEN, _EMBED), jnp.float32),
            [pltpu.SemaphoreType.DMA] * _NSLOT,
            [pltpu.SemaphoreType.DMA] * _NIDX,
            [pltpu.SemaphoreType.DMA] * _NSLOT,
        ],
    )(_emb_body)
    return f(xf, tok128, pos_table)


def kernel(x, token_table, pos_table):
    xf = x.astype(jnp.int32).reshape(_BATCH * _MAXLEN)
    tok128 = jnp.pad(token_table, ((0, 0), (0, 2 * _EMBED - token_table.shape[1])))
    out = _emb(xf, tok128, pos_table)
    return out.reshape(_BATCH, _MAXLEN, _EMBED)
